# Initial kernel scaffold; baseline (speedup 1.0000x reference)
import jax
import jax.numpy as jnp
from jax.experimental import pallas as pl


def kernel(head_e, rel_idx, tail_e, rel_embedding):
    raise NotImplementedError("write your pallas kernel here")


# speedup vs baseline: 1.2138x; 1.2138x over previous
import functools

import jax
import jax.numpy as jnp
from jax import lax
from jax.experimental import pallas as pl
from jax.experimental.pallas import tpu as pltpu
from jax.experimental.pallas import tpu_sc as plsc

BATCH = 16384
EMBED_DIM = 128
NUM_CORES = 2
NUM_SUBCORES = 16
NUM_WORKERS = NUM_CORES * NUM_SUBCORES
ROWS_PER_WORKER = BATCH // NUM_WORKERS
CHUNK = 128
NUM_CHUNKS = ROWS_PER_WORKER // CHUNK
LANES = 16
DCHUNKS = EMBED_DIM // LANES


def _distmult_body(head_hbm, idx_hbm, tail_hbm, rel_hbm, out_hbm,
                   idx_v, h_v, t_v, r_v, out_v, sem):
    wid = lax.axis_index("s") * NUM_CORES + lax.axis_index("c")
    base = wid * ROWS_PER_WORKER
    lane_iota = lax.iota(jnp.int32, LANES)

    def chunk_body(ci, carry):
        cbase = base + ci * CHUNK
        pltpu.sync_copy(idx_hbm.at[pl.ds(cbase, CHUNK)], idx_v)
        gather = pltpu.async_copy(rel_hbm.at[idx_v], r_v, sem)
        pltpu.sync_copy(head_hbm.at[pl.ds(cbase, CHUNK)], h_v)
        pltpu.sync_copy(tail_hbm.at[pl.ds(cbase, CHUNK)], t_v)
        gather.wait()

        def group_body(g, carry2):
            out_acc = jnp.zeros((LANES,), jnp.float32)
            for j in range(LANES):
                row = g * LANES + j
                acc = jnp.zeros((LANES,), jnp.float32)
                for c in range(DCHUNKS):
                    sl = pl.ds(c * LANES, LANES)
                    acc = acc + (h_v[row, sl] * r_v[row, sl]) * t_v[row, sl]
                s = jnp.sum(acc)
                out_acc = jnp.where(lane_iota == j, s, out_acc)
            out_v[pl.ds(g * LANES, LANES)] = out_acc
            return carry2

        lax.fori_loop(0, CHUNK // LANES, group_body, 0)
        pltpu.sync_copy(out_v, out_hbm.at[pl.ds(cbase, CHUNK)])
        return carry

    lax.fori_loop(0, NUM_CHUNKS, chunk_body, 0)


@jax.jit
def _distmult_sc(head_e, rel_idx, tail_e, rel_embedding):
    mesh = plsc.VectorSubcoreMesh(core_axis_name="c", subcore_axis_name="s")
    kern = functools.partial(
        pl.kernel,
        mesh=mesh,
        compiler_params=pltpu.CompilerParams(needs_layout_passes=False),
        out_type=jax.ShapeDtypeStruct((BATCH,), jnp.float32),
        scratch_types=[
            pltpu.VMEM((CHUNK,), jnp.int32),
            pltpu.VMEM((CHUNK, EMBED_DIM), jnp.float32),
            pltpu.VMEM((CHUNK, EMBED_DIM), jnp.float32),
            pltpu.VMEM((CHUNK, EMBED_DIM), jnp.float32),
            pltpu.VMEM((CHUNK,), jnp.float32),
            pltpu.SemaphoreType.DMA,
        ],
    )(_distmult_body)
    return kern(head_e, rel_idx, tail_e, rel_embedding)


def kernel(head_e, rel_idx, tail_e, rel_embedding):
    return _distmult_sc(head_e, rel_idx.astype(jnp.int32), tail_e,
                        rel_embedding)
